# Initial kernel scaffold; baseline (speedup 1.0000x reference)
import jax
import jax.numpy as jnp
from jax.experimental import pallas as pl


def kernel(embs, edge_index, batch_size):
    raise NotImplementedError("write your pallas kernel here")


# speedup vs baseline: 3.6044x; 3.6044x over previous
import jax
import jax.numpy as jnp
from jax.experimental import pallas as pl

_BLOCK_ROWS = 1000


_INV_SQRT2 = 0.7071067811865476


def _gelu_erf(x):
    return 0.5 * x * (1.0 + jax.lax.erf(x * _INV_SQRT2))


def _double_gelu_kernel(x_ref, o_ref):
    o_ref[...] = _gelu_erf(_gelu_erf(x_ref[...]))


def kernel(embs, edge_index, batch_size):
    n, d = embs.shape
    grid = (n // _BLOCK_ROWS,)
    return pl.pallas_call(
        _double_gelu_kernel,
        grid=grid,
        in_specs=[pl.BlockSpec((_BLOCK_ROWS, d), lambda i: (i, 0))],
        out_specs=pl.BlockSpec((_BLOCK_ROWS, d), lambda i: (i, 0)),
        out_shape=jax.ShapeDtypeStruct((n, d), embs.dtype),
    )(embs)


# speedup vs baseline: 4.3835x; 1.2162x over previous
import jax
import jax.numpy as jnp
from jax.experimental import pallas as pl

_BLOCK_ROWS = 2000


_INV_SQRT2 = 0.7071067811865476


def _gelu_erf(x):
    return 0.5 * x * (1.0 + jax.lax.erf(x * _INV_SQRT2))


def _double_gelu_kernel(x_ref, o_ref):
    o_ref[...] = _gelu_erf(_gelu_erf(x_ref[...]))


def kernel(embs, edge_index, batch_size):
    n, d = embs.shape
    grid = (n // _BLOCK_ROWS,)
    return pl.pallas_call(
        _double_gelu_kernel,
        grid=grid,
        in_specs=[pl.BlockSpec((_BLOCK_ROWS, d), lambda i: (i, 0))],
        out_specs=pl.BlockSpec((_BLOCK_ROWS, d), lambda i: (i, 0)),
        out_shape=jax.ShapeDtypeStruct((n, d), embs.dtype),
    )(embs)


# speedup vs baseline: 5.5635x; 1.2692x over previous
import jax
import jax.numpy as jnp
from jax.experimental import pallas as pl

_BLOCK_ROWS = 5000


_INV_SQRT2 = 0.7071067811865476


def _gelu_erf(x):
    return 0.5 * x * (1.0 + jax.lax.erf(x * _INV_SQRT2))


def _double_gelu_kernel(x_ref, o_ref):
    o_ref[...] = _gelu_erf(_gelu_erf(x_ref[...]))


def kernel(embs, edge_index, batch_size):
    n, d = embs.shape
    grid = (n // _BLOCK_ROWS,)
    return pl.pallas_call(
        _double_gelu_kernel,
        grid=grid,
        in_specs=[pl.BlockSpec((_BLOCK_ROWS, d), lambda i: (i, 0))],
        out_specs=pl.BlockSpec((_BLOCK_ROWS, d), lambda i: (i, 0)),
        out_shape=jax.ShapeDtypeStruct((n, d), embs.dtype),
    )(embs)
